# baseline (speedup 1.0000x reference)
import functools
import math

import jax
import jax.numpy as jnp
import numpy as np
from jax import lax
from jax.experimental import pallas as pl
from jax.experimental.pallas import tpu as pltpu
from jax.experimental.pallas import tpu_sc as plsc

MAX_LEN = 5000
DIM = 1024
N = 16384
B = 8
ZERO_ROW = MAX_LEN
SCALE = math.sqrt(DIM)

NUM_CORES = 2
NUM_SUBCORES = 16
NUM_WORKERS = NUM_CORES * NUM_SUBCORES
ROWS_PER_WORKER = N // NUM_WORKERS
CHUNK = 32
NUM_CHUNKS = ROWS_PER_WORKER // CHUNK
LANES = 16


def _pe_table() -> np.ndarray:
    pe = np.zeros((MAX_LEN + 1, DIM), dtype=np.float32)
    position = np.arange(0, MAX_LEN, dtype=np.float32)[:, None]
    div_term = np.exp(
        np.arange(0, DIM, 2, dtype=np.float32) * -(math.log(10000.0) / DIM))
    pe[:MAX_LEN, 0::2] = np.sin(position * div_term)
    pe[:MAX_LEN, 1::2] = np.cos(position * div_term)
    return pe


_PE = _pe_table()


def _index_body(mask_ref, out_ref):
    m = mask_ref[...]
    x = m
    for sh in (1, 2, 4, 8, 16, 32, 64):
        x = x + jnp.concatenate(
            [jnp.zeros((128, sh), jnp.int32), x[:, :-sh]], axis=1)
    row_tot = x[:, 127:128]
    y = row_tot
    for sh in (1, 2, 4, 8, 16, 32, 64):
        y = y + jnp.concatenate(
            [jnp.zeros((sh, 1), jnp.int32), y[:-sh, :]], axis=0)
    cs = x + (y - row_tot)
    rank = cs - 1
    col = lax.broadcasted_iota(jnp.int32, (128, 128), 1)
    bmod = col & 7
    cums = []
    running = jnp.zeros((), jnp.int32)
    starts = []
    for b in range(B):
        sl_b = jnp.sum(jnp.where(bmod == b, m, 0))
        starts.append(running)
        running = running + sl_b
        cums.append(running)
    batch = jnp.zeros((128, 128), jnp.int32)
    for b in range(B):
        batch = batch + (rank >= cums[b]).astype(jnp.int32)
    batch = jnp.minimum(batch, B - 1)
    start_sel = jnp.zeros((128, 128), jnp.int32)
    for b in range(B):
        start_sel = start_sel + jnp.where(batch == b, starts[b], 0)
    pos = rank - start_sel
    out_ref[...] = jnp.where(m > 0, pos, ZERO_ROW)


def _row_indices(mask_i32):
    return pl.pallas_call(
        _index_body,
        out_shape=jax.ShapeDtypeStruct((128, 128), jnp.int32),
    )(mask_i32)


def _sc_body(emb_hbm, idx_hbm, pe_hbm, out_hbm, ebuf, pbuf, ibuf, sem):
    wid = lax.axis_index("s") * NUM_CORES + lax.axis_index("c")
    base = wid * ROWS_PER_WORKER

    def chunk_fn(ch, carry):
        row0 = base + ch * CHUNK
        pltpu.sync_copy(emb_hbm.at[pl.ds(row0, CHUNK)], ebuf)
        pltpu.sync_copy(idx_hbm.at[pl.ds(row0, CHUNK)], ibuf)
        pltpu.async_copy(pe_hbm.at[ibuf], pbuf, sem).wait()

        def row_fn(r, c2):
            for c0 in range(0, DIM, LANES):
                e = ebuf[r, pl.ds(c0, LANES)]
                p = pbuf[r, pl.ds(c0, LANES)]
                ebuf[r, pl.ds(c0, LANES)] = e * SCALE + p
            return c2

        lax.fori_loop(0, CHUNK, row_fn, 0)
        pltpu.sync_copy(ebuf, out_hbm.at[pl.ds(row0, CHUNK)])
        return carry

    lax.fori_loop(0, NUM_CHUNKS, chunk_fn, 0)


@functools.cache
def _sc_apply():
    return pl.kernel(
        _sc_body,
        mesh=plsc.VectorSubcoreMesh(core_axis_name="c", subcore_axis_name="s"),
        out_type=jax.ShapeDtypeStruct((N, DIM), jnp.float32),
        scratch_types=[
            pltpu.VMEM((CHUNK, DIM), jnp.float32),
            pltpu.VMEM((CHUNK, DIM), jnp.float32),
            pltpu.VMEM((CHUNK,), jnp.int32),
            pltpu.SemaphoreType.DMA,
        ],
    )


def kernel(emb, position_mask):
    mask_i32 = position_mask.reshape(128, 128).astype(jnp.int32)
    idx = _row_indices(mask_i32).reshape(-1)
    emb_flat = emb.reshape(N, DIM)
    out_flat = _sc_apply()(emb_flat, idx, jnp.asarray(_PE))
    return out_flat.reshape(emb.shape)
